# Initial kernel scaffold; baseline (speedup 1.0000x reference)
import functools

import jax
import jax.numpy as jnp
from jax import lax
from jax.experimental import pallas as pl
from jax.experimental.pallas import tpu as pltpu
from jax.experimental.pallas import tpu_sc as plsc

B = 16384
L = 20
D = 32
NC = 2
NS = 16
NW = NC * NS
SPW = B // NW
CH = 64
NCH = SPW // CH
RPC = CH * L
KCH = RPC // 128


def _sc_body(p2, c2, g2, l1, Wp, Wc, Wg, Wl, out,
             idx_v, rows_v, sum_v, lidx_v, lrows_v, sem):
    cid = lax.axis_index("c")
    sid = lax.axis_index("s")
    wid = sid * NC + cid
    base = wid * SPW

    def do_chunk(ch, carry):
        g0 = base + ch * CH
        irow0 = g0 * (L // 4) // 32

        def do_table(tbl_hbm, idx2_hbm, col0):
            pltpu.sync_copy(idx2_hbm.at[pl.ds(irow0, KCH)], idx_v)
            cps = [
                pltpu.async_copy(tbl_hbm.at[idx_v.at[k]],
                                 rows_v.at[pl.ds(k * 128, 128)], sem)
                for k in range(KCH)
            ]
            for cp in cps:
                cp.wait()

            def sbody(s, c):
                r0 = s * L
                acc0 = jnp.zeros((16,), jnp.float32)
                acc1 = jnp.zeros((16,), jnp.float32)
                for j in range(L):
                    acc0 = acc0 + rows_v[r0 + j, pl.ds(0, 16)]
                    acc1 = acc1 + rows_v[r0 + j, pl.ds(16, 16)]
                sum_v[s, pl.ds(0, 16)] = acc0
                sum_v[s, pl.ds(16, 16)] = acc1
                return c

            lax.fori_loop(0, CH, sbody, 0)
            pltpu.sync_copy(sum_v, out.at[pl.ds(g0, CH), pl.ds(col0, 32)])

        do_table(Wp, p2, 0)
        do_table(Wc, c2, 32)
        do_table(Wg, g2, 64)

        pltpu.sync_copy(l1.at[pl.ds(g0, CH)], lidx_v)
        pltpu.async_copy(Wl.at[lidx_v], lrows_v, sem).wait()
        pltpu.sync_copy(lrows_v, out.at[pl.ds(g0, CH), pl.ds(96, 32)])
        return carry

    lax.fori_loop(0, NCH, do_chunk, 0)


@jax.jit
def _sc_call(p2, c2, g2, l1, Wp, Wc, Wg, Wl):
    mesh = plsc.VectorSubcoreMesh(core_axis_name="c", subcore_axis_name="s")
    return pl.kernel(
        _sc_body,
        out_type=jax.ShapeDtypeStruct((B, 4 * D), jnp.float32),
        mesh=mesh,
        scratch_types=[
            pltpu.VMEM((KCH, 128), jnp.int32),
            pltpu.VMEM((RPC, D), jnp.float32),
            pltpu.VMEM((CH, D), jnp.float32),
            pltpu.VMEM((CH,), jnp.int32),
            pltpu.VMEM((CH, D), jnp.float32),
            pltpu.SemaphoreType.DMA,
        ],
    )(p2, c2, g2, l1, Wp, Wc, Wg, Wl)


BLK = 2048


def _tc_body(ctx_ref, p_ref, c_ref, g_ref, r_ref,
             W1_ref, b1_ref, W2_ref, b2_ref, W3_ref, b3_ref,
             out_ref, wpop_ref, wvote_ref):
    f32 = jnp.float32

    def den(iref):
        m = (iref[...] != 0).astype(f32)
        cnt = jnp.maximum(jnp.sum(m, axis=1, keepdims=True), 1.0)
        return jnp.broadcast_to(cnt, (BLK, D))

    denom = jnp.concatenate(
        [den(p_ref), den(c_ref), den(g_ref), jnp.ones((BLK, D), f32)], axis=1)
    ctx = ctx_ref[...] / denom
    h = jnp.maximum(jnp.dot(ctx, W1_ref[...], preferred_element_type=f32)
                    + b1_ref[...], 0.0)
    h = jnp.maximum(jnp.dot(h, W2_ref[...], preferred_element_type=f32)
                    + b2_ref[...], 0.0)
    prm = jnp.dot(h, W3_ref[...], preferred_element_type=f32) + b3_ref[...]
    w_pop = prm[:, 0:1]
    w_vote = prm[:, 1:2]
    b_pop = prm[:, 2:3]
    b_vote = prm[:, 3:4]
    rr = r_ref[...]
    out_ref[...] = jnp.concatenate(
        [w_pop * rr + b_pop, w_vote * rr + b_vote], axis=1)
    wpop_ref[...] = w_pop
    wvote_ref[...] = w_vote


@jax.jit
def _tc_call(ctx, p, c, g, r, W1, b1, W2, b2, W3, b3):
    grid = (B // BLK,)
    bs_row = lambda width: pl.BlockSpec((BLK, width), lambda i: (i, 0))
    bs_full = lambda a, b: pl.BlockSpec((a, b), lambda i: (0, 0))
    return pl.pallas_call(
        _tc_body,
        grid=grid,
        in_specs=[
            bs_row(4 * D), bs_row(L), bs_row(L), bs_row(L), bs_row(1),
            bs_full(4 * D, 16), bs_full(1, 16),
            bs_full(16, 16), bs_full(1, 16),
            bs_full(16, 4), bs_full(1, 4),
        ],
        out_specs=[bs_row(2), bs_row(1), bs_row(1)],
        out_shape=[
            jax.ShapeDtypeStruct((B, 2), jnp.float32),
            jax.ShapeDtypeStruct((B, 1), jnp.float32),
            jax.ShapeDtypeStruct((B, 1), jnp.float32),
        ],
    )(ctx, p, c, g, r, W1, b1, W2, b2, W3, b3)


def kernel(r, p, c, g, l, W_prod, W_country, W_genre, W_lang,
           W1, b1, W2, b2, W3, b3):
    i32 = jnp.int32
    p32 = p.astype(i32)
    c32 = c.astype(i32)
    g32 = g.astype(i32)
    l32 = l.astype(i32)
    ctx = _sc_call(p32.reshape(-1, 128), c32.reshape(-1, 128),
                   g32.reshape(-1, 128), l32,
                   W_prod, W_country, W_genre, W_lang)
    out, w_pop, w_vote = _tc_call(
        ctx, p32, c32, g32, r,
        W1, b1.reshape(1, 16), W2, b2.reshape(1, 16), W3, b3.reshape(1, 4))
    return out, w_pop, w_vote


# speedup vs baseline: 6.1385x; 6.1385x over previous
import functools

import jax
import jax.numpy as jnp
from jax import lax
from jax.experimental import pallas as pl
from jax.experimental.pallas import tpu as pltpu
from jax.experimental.pallas import tpu_sc as plsc

B = 16384
L = 20
D = 32
NC = 2
NS = 16
NW = NC * NS
SPW = B // NW
CH = 64
NCH = SPW // CH
RPC = CH * L
KCH = RPC // 128


def _sc_body(p1, c1, g1, l1, Wp, Wc, Wg, Wl, out,
             idx_v, rows_v, sum_v, lidx_v, lrows_v, sem):
    cid = lax.axis_index("c")
    sid = lax.axis_index("s")
    wid = sid * NC + cid
    base = wid * SPW

    def do_chunk(ch, carry):
        g0 = base + ch * CH

        def do_table(tbl_hbm, idx1_hbm, col0):
            pltpu.sync_copy(idx1_hbm.at[pl.ds(g0 * L, RPC)], idx_v)
            cps = [
                pltpu.async_copy(tbl_hbm.at[idx_v.at[pl.ds(k * 128, 128)]],
                                 rows_v.at[pl.ds(k * 128, 128)], sem)
                for k in range(KCH)
            ]
            for cp in cps:
                cp.wait()

            def sbody(s, c):
                r0 = s * L
                acc0 = jnp.zeros((16,), jnp.float32)
                acc1 = jnp.zeros((16,), jnp.float32)
                for j in range(L):
                    acc0 = acc0 + rows_v[r0 + j, pl.ds(0, 16)]
                    acc1 = acc1 + rows_v[r0 + j, pl.ds(16, 16)]
                sum_v[s, pl.ds(col0, 16)] = acc0
                sum_v[s, pl.ds(col0 + 16, 16)] = acc1
                return c

            lax.fori_loop(0, CH, sbody, 0)

        do_table(Wp, p1, 0)
        do_table(Wc, c1, 32)
        do_table(Wg, g1, 64)

        pltpu.sync_copy(l1.at[pl.ds(g0, CH)], lidx_v)
        pltpu.async_copy(Wl.at[lidx_v], lrows_v, sem).wait()

        def lbody(s, c):
            sum_v[s, pl.ds(96, 16)] = lrows_v[s, pl.ds(0, 16)]
            sum_v[s, pl.ds(112, 16)] = lrows_v[s, pl.ds(16, 16)]
            return c

        lax.fori_loop(0, CH, lbody, 0)
        pltpu.sync_copy(sum_v, out.at[pl.ds(g0, CH)])
        return carry

    lax.fori_loop(0, NCH, do_chunk, 0)


@jax.jit
def _sc_call(p1, c1, g1, l1, Wp, Wc, Wg, Wl):
    mesh = plsc.VectorSubcoreMesh(core_axis_name="c", subcore_axis_name="s")
    return pl.kernel(
        _sc_body,
        out_type=jax.ShapeDtypeStruct((B, 4 * D), jnp.float32),
        mesh=mesh,
        compiler_params=pltpu.CompilerParams(use_tc_tiling_on_sc=False),
        scratch_types=[
            pltpu.VMEM((RPC,), jnp.int32),
            pltpu.VMEM((RPC, D), jnp.float32),
            pltpu.VMEM((CH, 4 * D), jnp.float32),
            pltpu.VMEM((CH,), jnp.int32),
            pltpu.VMEM((CH, D), jnp.float32),
            pltpu.SemaphoreType.DMA,
        ],
    )(p1, c1, g1, l1, Wp, Wc, Wg, Wl)


BLK = 2048


def _tc_body(ctx_ref, p_ref, c_ref, g_ref, r_ref,
             W1_ref, b1_ref, W2_ref, b2_ref, W3_ref, b3_ref,
             out_ref, wpop_ref, wvote_ref):
    f32 = jnp.float32

    def den(iref):
        m = (iref[...] != 0).astype(f32)
        cnt = jnp.maximum(jnp.sum(m, axis=1, keepdims=True), 1.0)
        return jnp.broadcast_to(cnt, (BLK, D))

    denom = jnp.concatenate(
        [den(p_ref), den(c_ref), den(g_ref), jnp.ones((BLK, D), f32)], axis=1)
    ctx = ctx_ref[...] / denom
    h = jnp.maximum(jnp.dot(ctx, W1_ref[...], preferred_element_type=f32)
                    + b1_ref[...], 0.0)
    h = jnp.maximum(jnp.dot(h, W2_ref[...], preferred_element_type=f32)
                    + b2_ref[...], 0.0)
    prm = jnp.dot(h, W3_ref[...], preferred_element_type=f32) + b3_ref[...]
    w_pop = prm[:, 0:1]
    w_vote = prm[:, 1:2]
    b_pop = prm[:, 2:3]
    b_vote = prm[:, 3:4]
    rr = r_ref[...]
    out_ref[...] = jnp.concatenate(
        [w_pop * rr + b_pop, w_vote * rr + b_vote], axis=1)
    wpop_ref[...] = w_pop
    wvote_ref[...] = w_vote


@jax.jit
def _tc_call(ctx, p, c, g, r, W1, b1, W2, b2, W3, b3):
    grid = (B // BLK,)
    bs_row = lambda width: pl.BlockSpec((BLK, width), lambda i: (i, 0))
    bs_full = lambda a, b: pl.BlockSpec((a, b), lambda i: (0, 0))
    return pl.pallas_call(
        _tc_body,
        grid=grid,
        in_specs=[
            bs_row(4 * D), bs_row(L), bs_row(L), bs_row(L), bs_row(1),
            bs_full(4 * D, 16), bs_full(1, 16),
            bs_full(16, 16), bs_full(1, 16),
            bs_full(16, 4), bs_full(1, 4),
        ],
        out_specs=[bs_row(2), bs_row(1), bs_row(1)],
        out_shape=[
            jax.ShapeDtypeStruct((B, 2), jnp.float32),
            jax.ShapeDtypeStruct((B, 1), jnp.float32),
            jax.ShapeDtypeStruct((B, 1), jnp.float32),
        ],
    )(ctx, p, c, g, r, W1, b1, W2, b2, W3, b3)


def kernel(r, p, c, g, l, W_prod, W_country, W_genre, W_lang,
           W1, b1, W2, b2, W3, b3):
    i32 = jnp.int32
    p32 = p.astype(i32)
    c32 = c.astype(i32)
    g32 = g.astype(i32)
    l32 = l.astype(i32)
    ctx = _sc_call(p32.reshape(-1), c32.reshape(-1), g32.reshape(-1), l32,
                   W_prod, W_country, W_genre, W_lang)
    out, w_pop, w_vote = _tc_call(
        ctx, p32, c32, g32, r,
        W1, b1.reshape(1, 16), W2, b2.reshape(1, 16), W3, b3.reshape(1, 4))
    return out, w_pop, w_vote
